# baseline (device time: 129863 ns/iter reference)
import jax
import jax.numpy as jnp
from jax import lax
from jax.experimental import pallas as pl
from jax.experimental.pallas import tpu as pltpu

N_DEV = 4
SQ = 2048
SKV = 2048
HQ = 8
DH = 128
DM = HQ * DH
HALO = 128
NG = 32
SCALE = 0.08838834764831843
BQ = 256
WIN = BQ + 2 * HALO
NCAT = NG + HALO + SKV + HALO
NEG = -1e9

PW = DM + 2 * HQ


def _b(a):
    return a.astype(jnp.bfloat16)


def _dot_t(a, b):
    return lax.dot_general(a, b, (((1,), (1,)), ((), ())),
                           preferred_element_type=jnp.float32)


def _dot_n(a, b):
    return lax.dot_general(a, b, (((1,), (0,)), ((), ())),
                           preferred_element_type=jnp.float32)


def kernel(x, Wq, K_ext, V_ext, Wo):
    def body(x_ref, wq_ref, k_ref, v_ref, wo_ref, out_ref,
             kcat, vcat, q_sc, qg_rec, part_snd, part_rec,
             ssem, rsem, lsem, exit_sem):
        my = lax.axis_index("i")

        def rdma(src, dst, s_slot, r_slot, target):
            return pltpu.make_async_remote_copy(
                src_ref=src, dst_ref=dst,
                send_sem=ssem.at[s_slot], recv_sem=rsem.at[r_slot],
                device_id=(target,), device_id_type=pl.DeviceIdType.MESH,
            )

        bar = pltpu.get_barrier_semaphore()
        for k in (1, 2, 3):
            pl.semaphore_signal(bar, inc=1, device_id=((my + k) % N_DEV,),
                                device_id_type=pl.DeviceIdType.MESH)
        pl.semaphore_wait(bar, N_DEV - 1)

        @pl.when(my > 0)
        def _():
            rdma(k_ref.at[0, pl.ds(0, HALO)],
                 kcat.at[pl.ds(NG + HALO + SKV, HALO)], 0, 2, my - 1).start()
            rdma(v_ref.at[0, pl.ds(0, HALO)],
                 vcat.at[pl.ds(NG + HALO + SKV, HALO)], 1, 3, my - 1).start()

        @pl.when(my < N_DEV - 1)
        def _():
            rdma(k_ref.at[0, pl.ds(SKV - HALO, HALO)],
                 kcat.at[pl.ds(NG, HALO)], 2, 0, my + 1).start()
            rdma(v_ref.at[0, pl.ds(SKV - HALO, HALO)],
                 vcat.at[pl.ds(NG, HALO)], 3, 1, my + 1).start()

        @pl.when(my == 0)
        def _():
            for j, t in enumerate((1, 2, 3)):
                rdma(k_ref.at[0, pl.ds(0, NG)], kcat.at[pl.ds(0, NG)],
                     4 + j, 4, t).start()
                rdma(v_ref.at[0, pl.ds(0, NG)], vcat.at[pl.ds(0, NG)],
                     7 + j, 5, t).start()

        cp_k = pltpu.make_async_copy(
            k_ref.at[0], kcat.at[pl.ds(NG + HALO, SKV)], lsem.at[0])
        cp_v = pltpu.make_async_copy(
            v_ref.at[0], vcat.at[pl.ds(NG + HALO, SKV)], lsem.at[1])
        cp_k.start()
        cp_v.start()

        @pl.when(my == 0)
        def _():
            gk = pltpu.make_async_copy(
                k_ref.at[0, pl.ds(0, NG)], kcat.at[pl.ds(0, NG)], lsem.at[2])
            gv = pltpu.make_async_copy(
                v_ref.at[0, pl.ds(0, NG)], vcat.at[pl.ds(0, NG)], lsem.at[3])
            gk.start()
            gv.start()
            gk.wait()
            gv.wait()

        q_sc[...] = _dot_n(_b(x_ref[0]), _b(wq_ref[...]))

        @pl.when(my == 0)
        def _():
            for j, t in enumerate((1, 2, 3)):
                rdma(q_sc.at[pl.ds(0, NG)], qg_rec, 10 + j, 6, t).start()

        @pl.when(my > 0)
        def _():
            rdma(k_ref.at[0, pl.ds(SKV - HALO, HALO)],
                 kcat.at[pl.ds(NG, HALO)], 2, 0, my).wait_recv()
            rdma(v_ref.at[0, pl.ds(SKV - HALO, HALO)],
                 vcat.at[pl.ds(NG, HALO)], 3, 1, my).wait_recv()
            rdma(k_ref.at[0, pl.ds(0, NG)], kcat.at[pl.ds(0, NG)],
                 4, 4, my).wait_recv()
            rdma(v_ref.at[0, pl.ds(0, NG)], vcat.at[pl.ds(0, NG)],
                 7, 5, my).wait_recv()
            rdma(q_sc.at[pl.ds(0, NG)], qg_rec, 10, 6, my).wait_recv()

        @pl.when(my < N_DEV - 1)
        def _():
            rdma(k_ref.at[0, pl.ds(0, HALO)],
                 kcat.at[pl.ds(NG + HALO + SKV, HALO)], 0, 2, my).wait_recv()
            rdma(v_ref.at[0, pl.ds(0, HALO)],
                 vcat.at[pl.ds(NG + HALO + SKV, HALO)], 1, 3, my).wait_recv()

        cp_k.wait()
        cp_v.wait()

        is0 = my == 0
        for h in range(HQ):
            qg_h = jnp.where(is0,
                             q_sc[pl.ds(0, NG), pl.ds(h * DH, DH)],
                             qg_rec[:, pl.ds(h * DH, DH)])
            kl = kcat[pl.ds(NG + HALO, SKV), h, :]
            vl = vcat[pl.ds(NG + HALO, SKV), h, :]
            s = _dot_t(_b(qg_h), _b(kl)) * SCALE
            m = jnp.max(s, axis=1, keepdims=True)
            p = jnp.exp(s - m)
            l = jnp.sum(p, axis=1, keepdims=True)
            o = _dot_n(_b(p), _b(vl))
            part_snd[pl.ds(0, NG), pl.ds(h * DH, DH)] = o
            part_snd[pl.ds(0, NG), pl.ds(DM + h, 1)] = m
            part_snd[pl.ds(0, NG), pl.ds(DM + HQ + h, 1)] = l

        for t in (1, 2, 3):
            @pl.when(my == t)
            def _():
                rdma(part_snd, part_rec.at[t - 1], 13, 6 + t, 0).start()

        def block_body(b, carry):
            qo = b * BQ
            r_iota = lax.broadcasted_iota(jnp.int32, (BQ, WIN), 0)
            w_iota = lax.broadcasted_iota(jnp.int32, (BQ, WIN), 1)
            ki = my * SKV + (qo - HALO) + w_iota
            valid = (ki >= 0) & (ki < N_DEV * SKV)
            band = (w_iota >= r_iota) & (w_iota <= r_iota + 2 * HALO)
            bias_w = jnp.where(valid & (band | (ki < NG)), 0.0, NEG)
            bias_g = jnp.where((my > 0) | (qo > 0), 0.0, NEG)
            for h in range(HQ):
                qb = q_sc[pl.ds(qo, BQ), pl.ds(h * DH, DH)]
                kw = kcat[pl.ds(NG + qo, WIN), h, :]
                kg = kcat[pl.ds(0, NG), h, :]
                s_w = _dot_t(_b(qb), _b(kw)) * SCALE + bias_w
                s_g = _dot_t(_b(qb), _b(kg)) * SCALE + bias_g
                s = jnp.concatenate([s_g, s_w], axis=1)
                mx = jnp.max(s, axis=1, keepdims=True)
                p = jnp.exp(s - mx)
                lsum = jnp.sum(p, axis=1, keepdims=True)
                vw = vcat[pl.ds(NG + qo, WIN), h, :]
                vg = vcat[pl.ds(0, NG), h, :]
                o = _dot_n(_b(p[:, NG:]), _b(vw)) + _dot_n(_b(p[:, :NG]), _b(vg))
                out_ref[0, pl.ds(qo, BQ), pl.ds(h * DH, DH)] = o / lsum
            return carry

        lax.fori_loop(0, SQ // BQ, block_body, 0)

        @pl.when(my == 0)
        def _():
            for j in (1, 2, 3):
                rdma(part_snd, part_rec.at[j - 1], 13, 6 + j, my).wait_recv()
            for h in range(HQ):
                ms, ls, os_ = [], [], []
                ms.append(part_snd[pl.ds(0, NG), pl.ds(DM + h, 1)])
                ls.append(part_snd[pl.ds(0, NG), pl.ds(DM + HQ + h, 1)])
                os_.append(part_snd[pl.ds(0, NG), pl.ds(h * DH, DH)])
                for j in range(3):
                    ms.append(part_rec[j, pl.ds(0, NG), pl.ds(DM + h, 1)])
                    ls.append(part_rec[j, pl.ds(0, NG), pl.ds(DM + HQ + h, 1)])
                    os_.append(part_rec[j, pl.ds(0, NG), pl.ds(h * DH, DH)])
                mm = jnp.maximum(jnp.maximum(ms[0], ms[1]),
                                 jnp.maximum(ms[2], ms[3]))
                lt = jnp.zeros_like(ls[0])
                ot = jnp.zeros_like(os_[0])
                for mj, lj, oj in zip(ms, ls, os_):
                    c = jnp.exp(mj - mm)
                    lt = lt + c * lj
                    ot = ot + c * oj
                out_ref[0, pl.ds(0, NG), pl.ds(h * DH, DH)] = ot / lt

        out_ref[0] = _dot_n(_b(out_ref[0]), _b(wo_ref[...]))

        @pl.when(my > 0)
        def _():
            rdma(k_ref.at[0, pl.ds(0, HALO)],
                 kcat.at[pl.ds(NG + HALO + SKV, HALO)], 0, 2, my).wait_send()
            rdma(v_ref.at[0, pl.ds(0, HALO)],
                 vcat.at[pl.ds(NG + HALO + SKV, HALO)], 1, 3, my).wait_send()
            rdma(part_snd, part_rec.at[0], 13, 7, my).wait_send()

        @pl.when(my < N_DEV - 1)
        def _():
            rdma(k_ref.at[0, pl.ds(SKV - HALO, HALO)],
                 kcat.at[pl.ds(NG, HALO)], 2, 0, my).wait_send()
            rdma(v_ref.at[0, pl.ds(SKV - HALO, HALO)],
                 vcat.at[pl.ds(NG, HALO)], 3, 1, my).wait_send()

        @pl.when(my == 0)
        def _():
            for j in range(3):
                rdma(k_ref.at[0, pl.ds(0, NG)], kcat.at[pl.ds(0, NG)],
                     4 + j, 4, my).wait_send()
                rdma(v_ref.at[0, pl.ds(0, NG)], vcat.at[pl.ds(0, NG)],
                     7 + j, 5, my).wait_send()
                rdma(q_sc.at[pl.ds(0, NG)], qg_rec, 10 + j, 6, my).wait_send()

        for k in (1, 2, 3):
            pl.semaphore_signal(exit_sem, inc=1, device_id=((my + k) % N_DEV,),
                                device_id_type=pl.DeviceIdType.MESH)
        pl.semaphore_wait(exit_sem, N_DEV - 1)

    return pl.pallas_call(
        body,
        out_shape=jax.ShapeDtypeStruct((1, SQ, DM), jnp.float32),
        in_specs=[
            pl.BlockSpec(memory_space=pltpu.VMEM),
            pl.BlockSpec(memory_space=pltpu.VMEM),
            pl.BlockSpec(memory_space=pltpu.HBM),
            pl.BlockSpec(memory_space=pltpu.HBM),
            pl.BlockSpec(memory_space=pltpu.VMEM),
        ],
        out_specs=pl.BlockSpec(memory_space=pltpu.VMEM),
        scratch_shapes=[
            pltpu.VMEM((NCAT, HQ, DH), jnp.float32),
            pltpu.VMEM((NCAT, HQ, DH), jnp.float32),
            pltpu.VMEM((SQ, DM), jnp.float32),
            pltpu.VMEM((NG, DM), jnp.float32),
            pltpu.VMEM((NG, PW), jnp.float32),
            pltpu.VMEM((3, NG, PW), jnp.float32),
            pltpu.SemaphoreType.DMA((16,)),
            pltpu.SemaphoreType.DMA((10,)),
            pltpu.SemaphoreType.DMA((4,)),
            pltpu.SemaphoreType.REGULAR,
        ],
        compiler_params=pltpu.CompilerParams(
            collective_id=0, vmem_limit_bytes=120 * 1024 * 1024),
    )(x, Wq, K_ext, V_ext, Wo)


# device time: 97903 ns/iter; 1.3264x vs baseline; 1.3264x over previous
import jax
import jax.numpy as jnp
from jax import lax
from jax.experimental import pallas as pl
from jax.experimental.pallas import tpu as pltpu

N_DEV = 4
SQ = 2048
SKV = 2048
HQ = 8
DH = 128
DM = HQ * DH
HALO = 128
NG = 32
SCALE = 0.08838834764831843
BQ = 256
WIN = BQ + 2 * HALO
NCAT = NG + HALO + SKV + HALO
NEG = -1e9

PW = DM + 2 * HQ


def _b(a):
    return a.astype(jnp.bfloat16)


def _dot_t(a, b):
    return lax.dot_general(a, b, (((1,), (1,)), ((), ())),
                           preferred_element_type=jnp.float32)


def _dot_n(a, b):
    return lax.dot_general(a, b, (((1,), (0,)), ((), ())),
                           preferred_element_type=jnp.float32)


def kernel(x, Wq, K_ext, V_ext, Wo):
    def body(x_ref, wq_ref, k_ref, v_ref, wo_ref, out_ref,
             kcat, vcat, q_sc, qg_rec, part_snd, part_rec,
             ssem, rsem, lsem, exit_sem):
        my = lax.axis_index("i")

        def rdma(src, dst, s_slot, r_slot, target):
            return pltpu.make_async_remote_copy(
                src_ref=src, dst_ref=dst,
                send_sem=ssem.at[s_slot], recv_sem=rsem.at[r_slot],
                device_id=(target,), device_id_type=pl.DeviceIdType.MESH,
            )

        bar = pltpu.get_barrier_semaphore()
        for k in (1, 2, 3):
            pl.semaphore_signal(bar, inc=1, device_id=((my + k) % N_DEV,),
                                device_id_type=pl.DeviceIdType.MESH)
        pl.semaphore_wait(bar, N_DEV - 1)

        @pl.when(my > 0)
        def _():
            rdma(k_ref.at[0, pl.ds(0, HALO)],
                 kcat.at[pl.ds(NG + HALO + SKV, HALO)], 0, 2, my - 1).start()
            rdma(v_ref.at[0, pl.ds(0, HALO)],
                 vcat.at[pl.ds(NG + HALO + SKV, HALO)], 1, 3, my - 1).start()

        @pl.when(my < N_DEV - 1)
        def _():
            rdma(k_ref.at[0, pl.ds(SKV - HALO, HALO)],
                 kcat.at[pl.ds(NG, HALO)], 2, 0, my + 1).start()
            rdma(v_ref.at[0, pl.ds(SKV - HALO, HALO)],
                 vcat.at[pl.ds(NG, HALO)], 3, 1, my + 1).start()

        @pl.when(my == 0)
        def _():
            for j, t in enumerate((1, 2, 3)):
                rdma(k_ref.at[0, pl.ds(0, NG)], kcat.at[pl.ds(0, NG)],
                     4 + j, 4, t).start()
                rdma(v_ref.at[0, pl.ds(0, NG)], vcat.at[pl.ds(0, NG)],
                     7 + j, 5, t).start()

        cp_k = pltpu.make_async_copy(
            k_ref.at[0], kcat.at[pl.ds(NG + HALO, SKV)], lsem.at[0])
        cp_v = pltpu.make_async_copy(
            v_ref.at[0], vcat.at[pl.ds(NG + HALO, SKV)], lsem.at[1])
        cp_k.start()
        cp_v.start()

        @pl.when(my == 0)
        def _():
            gk = pltpu.make_async_copy(
                k_ref.at[0, pl.ds(0, NG)], kcat.at[pl.ds(0, NG)], lsem.at[2])
            gv = pltpu.make_async_copy(
                v_ref.at[0, pl.ds(0, NG)], vcat.at[pl.ds(0, NG)], lsem.at[3])
            gk.start()
            gv.start()
            gk.wait()
            gv.wait()

        q_sc[...] = _dot_n(x_ref[0], wq_ref[...])

        @pl.when(my == 0)
        def _():
            for j, t in enumerate((1, 2, 3)):
                rdma(q_sc.at[pl.ds(0, NG)], qg_rec, 10 + j, 6, t).start()

        @pl.when(my > 0)
        def _():
            rdma(k_ref.at[0, pl.ds(SKV - HALO, HALO)],
                 kcat.at[pl.ds(NG, HALO)], 2, 0, my).wait_recv()
            rdma(v_ref.at[0, pl.ds(SKV - HALO, HALO)],
                 vcat.at[pl.ds(NG, HALO)], 3, 1, my).wait_recv()
            rdma(k_ref.at[0, pl.ds(0, NG)], kcat.at[pl.ds(0, NG)],
                 4, 4, my).wait_recv()
            rdma(v_ref.at[0, pl.ds(0, NG)], vcat.at[pl.ds(0, NG)],
                 7, 5, my).wait_recv()
            rdma(q_sc.at[pl.ds(0, NG)], qg_rec, 10, 6, my).wait_recv()

        @pl.when(my < N_DEV - 1)
        def _():
            rdma(k_ref.at[0, pl.ds(0, HALO)],
                 kcat.at[pl.ds(NG + HALO + SKV, HALO)], 0, 2, my).wait_recv()
            rdma(v_ref.at[0, pl.ds(0, HALO)],
                 vcat.at[pl.ds(NG + HALO + SKV, HALO)], 1, 3, my).wait_recv()

        cp_k.wait()
        cp_v.wait()

        is0 = my == 0
        for h in range(HQ):
            qg_h = jnp.where(is0,
                             q_sc[pl.ds(0, NG), pl.ds(h * DH, DH)],
                             qg_rec[:, pl.ds(h * DH, DH)])
            kl = kcat[pl.ds(NG + HALO, SKV), h, :]
            vl = vcat[pl.ds(NG + HALO, SKV), h, :]
            s = _dot_t(qg_h, kl) * SCALE
            m = jnp.max(s, axis=1, keepdims=True)
            p = jnp.exp(s - m)
            l = jnp.sum(p, axis=1, keepdims=True)
            o = _dot_n(p, vl)
            part_snd[pl.ds(0, NG), pl.ds(h * DH, DH)] = o
            part_snd[pl.ds(0, NG), pl.ds(DM + h, 1)] = m
            part_snd[pl.ds(0, NG), pl.ds(DM + HQ + h, 1)] = l

        for t in (1, 2, 3):
            @pl.when(my == t)
            def _():
                rdma(part_snd, part_rec.at[t - 1], 13, 6 + t, 0).start()

        def block_body(b, carry):
            qo = b * BQ
            r_iota = lax.broadcasted_iota(jnp.int32, (BQ, WIN), 0)
            w_iota = lax.broadcasted_iota(jnp.int32, (BQ, WIN), 1)
            ki = my * SKV + (qo - HALO) + w_iota
            valid = (ki >= 0) & (ki < N_DEV * SKV)
            band = (w_iota >= r_iota) & (w_iota <= r_iota + 2 * HALO)
            bias_w = jnp.where(valid & (band | (ki < NG)), 0.0, NEG)
            bias_g = jnp.where((my > 0) | (qo > 0), 0.0, NEG)
            for h in range(HQ):
                qb = q_sc[pl.ds(qo, BQ), pl.ds(h * DH, DH)]
                kw = kcat[pl.ds(NG + qo, WIN), h, :]
                kg = kcat[pl.ds(0, NG), h, :]
                s_w = _dot_t(qb, kw) * SCALE + bias_w
                s_g = _dot_t(qb, kg) * SCALE + bias_g
                mx = jnp.maximum(jnp.max(s_w, axis=1, keepdims=True),
                                 jnp.max(s_g, axis=1, keepdims=True))
                p_w = jnp.exp(s_w - mx)
                p_g = jnp.exp(s_g - mx)
                lsum = (jnp.sum(p_w, axis=1, keepdims=True)
                        + jnp.sum(p_g, axis=1, keepdims=True))
                vw = vcat[pl.ds(NG + qo, WIN), h, :]
                vg = vcat[pl.ds(0, NG), h, :]
                o = _dot_n(p_w, vw) + _dot_n(p_g, vg)
                out_ref[0, pl.ds(qo, BQ), pl.ds(h * DH, DH)] = o / lsum
            return carry

        lax.fori_loop(0, SQ // BQ, block_body, 0)

        @pl.when(my == 0)
        def _():
            for j in (1, 2, 3):
                rdma(part_snd, part_rec.at[j - 1], 13, 6 + j, my).wait_recv()
            for h in range(HQ):
                ms, ls, os_ = [], [], []
                ms.append(part_snd[pl.ds(0, NG), pl.ds(DM + h, 1)])
                ls.append(part_snd[pl.ds(0, NG), pl.ds(DM + HQ + h, 1)])
                os_.append(part_snd[pl.ds(0, NG), pl.ds(h * DH, DH)])
                for j in range(3):
                    ms.append(part_rec[j, pl.ds(0, NG), pl.ds(DM + h, 1)])
                    ls.append(part_rec[j, pl.ds(0, NG), pl.ds(DM + HQ + h, 1)])
                    os_.append(part_rec[j, pl.ds(0, NG), pl.ds(h * DH, DH)])
                mm = jnp.maximum(jnp.maximum(ms[0], ms[1]),
                                 jnp.maximum(ms[2], ms[3]))
                lt = jnp.zeros_like(ls[0])
                ot = jnp.zeros_like(os_[0])
                for mj, lj, oj in zip(ms, ls, os_):
                    c = jnp.exp(mj - mm)
                    lt = lt + c * lj
                    ot = ot + c * oj
                out_ref[0, pl.ds(0, NG), pl.ds(h * DH, DH)] = ot / lt

        out_ref[0] = _dot_n(out_ref[0], wo_ref[...])

        @pl.when(my > 0)
        def _():
            rdma(k_ref.at[0, pl.ds(0, HALO)],
                 kcat.at[pl.ds(NG + HALO + SKV, HALO)], 0, 2, my).wait_send()
            rdma(v_ref.at[0, pl.ds(0, HALO)],
                 vcat.at[pl.ds(NG + HALO + SKV, HALO)], 1, 3, my).wait_send()
            rdma(part_snd, part_rec.at[0], 13, 7, my).wait_send()

        @pl.when(my < N_DEV - 1)
        def _():
            rdma(k_ref.at[0, pl.ds(SKV - HALO, HALO)],
                 kcat.at[pl.ds(NG, HALO)], 2, 0, my).wait_send()
            rdma(v_ref.at[0, pl.ds(SKV - HALO, HALO)],
                 vcat.at[pl.ds(NG, HALO)], 3, 1, my).wait_send()

        @pl.when(my == 0)
        def _():
            for j in range(3):
                rdma(k_ref.at[0, pl.ds(0, NG)], kcat.at[pl.ds(0, NG)],
                     4 + j, 4, my).wait_send()
                rdma(v_ref.at[0, pl.ds(0, NG)], vcat.at[pl.ds(0, NG)],
                     7 + j, 5, my).wait_send()
                rdma(q_sc.at[pl.ds(0, NG)], qg_rec, 10 + j, 6, my).wait_send()

        for k in (1, 2, 3):
            pl.semaphore_signal(exit_sem, inc=1, device_id=((my + k) % N_DEV,),
                                device_id_type=pl.DeviceIdType.MESH)
        pl.semaphore_wait(exit_sem, N_DEV - 1)

    return pl.pallas_call(
        body,
        out_shape=jax.ShapeDtypeStruct((1, SQ, DM), jnp.float32),
        in_specs=[
            pl.BlockSpec(memory_space=pltpu.VMEM),
            pl.BlockSpec(memory_space=pltpu.VMEM),
            pl.BlockSpec(memory_space=pltpu.HBM),
            pl.BlockSpec(memory_space=pltpu.HBM),
            pl.BlockSpec(memory_space=pltpu.VMEM),
        ],
        out_specs=pl.BlockSpec(memory_space=pltpu.VMEM),
        scratch_shapes=[
            pltpu.VMEM((NCAT, HQ, DH), jnp.float32),
            pltpu.VMEM((NCAT, HQ, DH), jnp.float32),
            pltpu.VMEM((SQ, DM), jnp.float32),
            pltpu.VMEM((NG, DM), jnp.float32),
            pltpu.VMEM((NG, PW), jnp.float32),
            pltpu.VMEM((3, NG, PW), jnp.float32),
            pltpu.SemaphoreType.DMA((16,)),
            pltpu.SemaphoreType.DMA((10,)),
            pltpu.SemaphoreType.DMA((4,)),
            pltpu.SemaphoreType.REGULAR,
        ],
        compiler_params=pltpu.CompilerParams(
            collective_id=0, vmem_limit_bytes=120 * 1024 * 1024),
    )(x, Wq, K_ext, V_ext, Wo)


# device time: 84545 ns/iter; 1.5360x vs baseline; 1.1580x over previous
import jax
import jax.numpy as jnp
from jax import lax
from jax.experimental import pallas as pl
from jax.experimental.pallas import tpu as pltpu

N_DEV = 4
SQ = 2048
SKV = 2048
HQ = 8
DH = 128
DM = HQ * DH
HALO = 128
NG = 32
SCALE = 0.08838834764831843
BQ = 256
WIN = BQ + 2 * HALO
NCAT = NG + HALO + SKV + HALO
NEG = -1e9

PW = DM + 2 * HQ


def _b(a):
    return a.astype(jnp.bfloat16)


def _dot_t(a, b):
    return lax.dot_general(a, b, (((1,), (1,)), ((), ())),
                           preferred_element_type=jnp.float32)


def _dot_n(a, b):
    return lax.dot_general(a, b, (((1,), (0,)), ((), ())),
                           preferred_element_type=jnp.float32)


def kernel(x, Wq, K_ext, V_ext, Wo):
    def body(x_ref, wq_ref, k_ref, v_ref, wo_ref, out_ref,
             kcat, vcat, q_sc, qg_rec, part_snd, part_rec,
             ssem, rsem, lsem, exit_sem):
        my = lax.axis_index("i")

        def rdma(src, dst, s_slot, r_slot, target):
            return pltpu.make_async_remote_copy(
                src_ref=src, dst_ref=dst,
                send_sem=ssem.at[s_slot], recv_sem=rsem.at[r_slot],
                device_id=(target,), device_id_type=pl.DeviceIdType.MESH,
            )

        bar = pltpu.get_barrier_semaphore()
        for k in (1, 2, 3):
            pl.semaphore_signal(bar, inc=1, device_id=((my + k) % N_DEV,),
                                device_id_type=pl.DeviceIdType.MESH)
        pl.semaphore_wait(bar, N_DEV - 1)

        @pl.when(my > 0)
        def _():
            rdma(k_ref.at[0, pl.ds(0, HALO)],
                 kcat.at[pl.ds(NG + HALO + SKV, HALO)], 0, 2, my - 1).start()
            rdma(v_ref.at[0, pl.ds(0, HALO)],
                 vcat.at[pl.ds(NG + HALO + SKV, HALO)], 1, 3, my - 1).start()

        @pl.when(my < N_DEV - 1)
        def _():
            rdma(k_ref.at[0, pl.ds(SKV - HALO, HALO)],
                 kcat.at[pl.ds(NG, HALO)], 2, 0, my + 1).start()
            rdma(v_ref.at[0, pl.ds(SKV - HALO, HALO)],
                 vcat.at[pl.ds(NG, HALO)], 3, 1, my + 1).start()

        @pl.when(my == 0)
        def _():
            for j, t in enumerate((1, 2, 3)):
                rdma(k_ref.at[0, pl.ds(0, NG)], kcat.at[pl.ds(0, NG)],
                     4 + j, 4, t).start()
                rdma(v_ref.at[0, pl.ds(0, NG)], vcat.at[pl.ds(0, NG)],
                     7 + j, 5, t).start()

        cp_k = pltpu.make_async_copy(
            k_ref.at[0], kcat.at[pl.ds(NG + HALO, SKV)], lsem.at[0])
        cp_v = pltpu.make_async_copy(
            v_ref.at[0], vcat.at[pl.ds(NG + HALO, SKV)], lsem.at[1])
        cp_k.start()
        cp_v.start()

        @pl.when(my == 0)
        def _():
            gk = pltpu.make_async_copy(
                k_ref.at[0, pl.ds(0, NG)], kcat.at[pl.ds(0, NG)], lsem.at[2])
            gv = pltpu.make_async_copy(
                v_ref.at[0, pl.ds(0, NG)], vcat.at[pl.ds(0, NG)], lsem.at[3])
            gk.start()
            gv.start()
            gk.wait()
            gv.wait()

        q_sc[...] = _dot_n(x_ref[0], wq_ref[...])

        @pl.when(my == 0)
        def _():
            for j, t in enumerate((1, 2, 3)):
                rdma(q_sc.at[pl.ds(0, NG)], qg_rec, 10 + j, 6, t).start()

        @pl.when(my > 0)
        def _():
            rdma(k_ref.at[0, pl.ds(SKV - HALO, HALO)],
                 kcat.at[pl.ds(NG, HALO)], 2, 0, my).wait_recv()
            rdma(v_ref.at[0, pl.ds(SKV - HALO, HALO)],
                 vcat.at[pl.ds(NG, HALO)], 3, 1, my).wait_recv()
            rdma(k_ref.at[0, pl.ds(0, NG)], kcat.at[pl.ds(0, NG)],
                 4, 4, my).wait_recv()
            rdma(v_ref.at[0, pl.ds(0, NG)], vcat.at[pl.ds(0, NG)],
                 7, 5, my).wait_recv()
            rdma(q_sc.at[pl.ds(0, NG)], qg_rec, 10, 6, my).wait_recv()

        @pl.when(my < N_DEV - 1)
        def _():
            rdma(k_ref.at[0, pl.ds(0, HALO)],
                 kcat.at[pl.ds(NG + HALO + SKV, HALO)], 0, 2, my).wait_recv()
            rdma(v_ref.at[0, pl.ds(0, HALO)],
                 vcat.at[pl.ds(NG + HALO + SKV, HALO)], 1, 3, my).wait_recv()

        cp_k.wait()
        cp_v.wait()

        is0 = my == 0
        for h in range(HQ):
            qg_h = jnp.where(is0,
                             q_sc[pl.ds(0, NG), pl.ds(h * DH, DH)],
                             qg_rec[:, pl.ds(h * DH, DH)])
            kl = kcat[pl.ds(NG + HALO, SKV), h, :]
            vl = vcat[pl.ds(NG + HALO, SKV), h, :]
            s = _dot_t(qg_h, kl) * SCALE
            m = jnp.max(s, axis=1, keepdims=True)
            p = jnp.exp(s - m)
            l = jnp.sum(p, axis=1, keepdims=True)
            o = _dot_n(p, vl)
            part_snd[pl.ds(0, NG), pl.ds(h * DH, DH)] = o
            part_snd[pl.ds(0, NG), pl.ds(DM + h, 1)] = m
            part_snd[pl.ds(0, NG), pl.ds(DM + HQ + h, 1)] = l

        for t in (1, 2, 3):
            @pl.when(my == t)
            def _():
                rdma(part_snd, part_rec.at[t - 1], 13, 6 + t, 0).start()

        def block_body(b, carry):
            qo = b * BQ
            r_iota = lax.broadcasted_iota(jnp.int32, (BQ, WIN), 0)
            w_iota = lax.broadcasted_iota(jnp.int32, (BQ, WIN), 1)
            ki = my * SKV + (qo - HALO) + w_iota
            valid = (ki >= 0) & (ki < N_DEV * SKV)
            band = (w_iota >= r_iota) & (w_iota <= r_iota + 2 * HALO)
            bias_w = jnp.where(valid & (band | (ki < NG)), 0.0, NEG)
            bias_g = jnp.where((my > 0) | (qo > 0), 0.0, NEG)
            for h in range(HQ):
                qb = q_sc[pl.ds(qo, BQ), pl.ds(h * DH, DH)]
                kw = kcat[pl.ds(NG + qo, WIN), h, :]
                kg = kcat[pl.ds(0, NG), h, :]
                s_w = _dot_t(qb, kw) * SCALE + bias_w
                s_g = _dot_t(qb, kg) * SCALE + bias_g
                p_w = jnp.exp(s_w)
                p_g = jnp.exp(s_g)
                lsum = (jnp.sum(p_w, axis=1, keepdims=True)
                        + jnp.sum(p_g, axis=1, keepdims=True))
                vw = vcat[pl.ds(NG + qo, WIN), h, :]
                vg = vcat[pl.ds(0, NG), h, :]
                o = _dot_n(p_w, vw) + _dot_n(p_g, vg)
                out_ref[0, pl.ds(qo, BQ), pl.ds(h * DH, DH)] = o / lsum
            return carry

        lax.fori_loop(0, SQ // BQ, block_body, 0)

        @pl.when(my == 0)
        def _():
            for j in (1, 2, 3):
                rdma(part_snd, part_rec.at[j - 1], 13, 6 + j, my).wait_recv()
            for h in range(HQ):
                ms, ls, os_ = [], [], []
                ms.append(part_snd[pl.ds(0, NG), pl.ds(DM + h, 1)])
                ls.append(part_snd[pl.ds(0, NG), pl.ds(DM + HQ + h, 1)])
                os_.append(part_snd[pl.ds(0, NG), pl.ds(h * DH, DH)])
                for j in range(3):
                    ms.append(part_rec[j, pl.ds(0, NG), pl.ds(DM + h, 1)])
                    ls.append(part_rec[j, pl.ds(0, NG), pl.ds(DM + HQ + h, 1)])
                    os_.append(part_rec[j, pl.ds(0, NG), pl.ds(h * DH, DH)])
                mm = jnp.maximum(jnp.maximum(ms[0], ms[1]),
                                 jnp.maximum(ms[2], ms[3]))
                lt = jnp.zeros_like(ls[0])
                ot = jnp.zeros_like(os_[0])
                for mj, lj, oj in zip(ms, ls, os_):
                    c = jnp.exp(mj - mm)
                    lt = lt + c * lj
                    ot = ot + c * oj
                out_ref[0, pl.ds(0, NG), pl.ds(h * DH, DH)] = ot / lt

        out_ref[0] = _dot_n(out_ref[0], wo_ref[...])

        @pl.when(my > 0)
        def _():
            rdma(k_ref.at[0, pl.ds(0, HALO)],
                 kcat.at[pl.ds(NG + HALO + SKV, HALO)], 0, 2, my).wait_send()
            rdma(v_ref.at[0, pl.ds(0, HALO)],
                 vcat.at[pl.ds(NG + HALO + SKV, HALO)], 1, 3, my).wait_send()
            rdma(part_snd, part_rec.at[0], 13, 7, my).wait_send()

        @pl.when(my < N_DEV - 1)
        def _():
            rdma(k_ref.at[0, pl.ds(SKV - HALO, HALO)],
                 kcat.at[pl.ds(NG, HALO)], 2, 0, my).wait_send()
            rdma(v_ref.at[0, pl.ds(SKV - HALO, HALO)],
                 vcat.at[pl.ds(NG, HALO)], 3, 1, my).wait_send()

        @pl.when(my == 0)
        def _():
            for j in range(3):
                rdma(k_ref.at[0, pl.ds(0, NG)], kcat.at[pl.ds(0, NG)],
                     4 + j, 4, my).wait_send()
                rdma(v_ref.at[0, pl.ds(0, NG)], vcat.at[pl.ds(0, NG)],
                     7 + j, 5, my).wait_send()
                rdma(q_sc.at[pl.ds(0, NG)], qg_rec, 10 + j, 6, my).wait_send()

        for k in (1, 2, 3):
            pl.semaphore_signal(exit_sem, inc=1, device_id=((my + k) % N_DEV,),
                                device_id_type=pl.DeviceIdType.MESH)
        pl.semaphore_wait(exit_sem, N_DEV - 1)

    return pl.pallas_call(
        body,
        out_shape=jax.ShapeDtypeStruct((1, SQ, DM), jnp.float32),
        in_specs=[
            pl.BlockSpec(memory_space=pltpu.VMEM),
            pl.BlockSpec(memory_space=pltpu.VMEM),
            pl.BlockSpec(memory_space=pltpu.HBM),
            pl.BlockSpec(memory_space=pltpu.HBM),
            pl.BlockSpec(memory_space=pltpu.VMEM),
        ],
        out_specs=pl.BlockSpec(memory_space=pltpu.VMEM),
        scratch_shapes=[
            pltpu.VMEM((NCAT, HQ, DH), jnp.float32),
            pltpu.VMEM((NCAT, HQ, DH), jnp.float32),
            pltpu.VMEM((SQ, DM), jnp.float32),
            pltpu.VMEM((NG, DM), jnp.float32),
            pltpu.VMEM((NG, PW), jnp.float32),
            pltpu.VMEM((3, NG, PW), jnp.float32),
            pltpu.SemaphoreType.DMA((16,)),
            pltpu.SemaphoreType.DMA((10,)),
            pltpu.SemaphoreType.DMA((4,)),
            pltpu.SemaphoreType.REGULAR,
        ],
        compiler_params=pltpu.CompilerParams(
            collective_id=0, vmem_limit_bytes=120 * 1024 * 1024),
    )(x, Wq, K_ext, V_ext, Wo)
